# baseline (speedup 1.0000x reference)
import dataclasses

import jax
import jax.numpy as jnp
from jax import lax
from jax.experimental import pallas as pl
from jax.experimental.pallas import tpu as pltpu
from jax.experimental.pallas import tpu_sc as plsc

_B = 16384
_D = 64
_L = 16
_NC = 2
_NS = 16
_NW = _NC * _NS
_BPW = _B // _NW
_W = 128
_NWIN = _BPW // _W
_NU = 1000000
_NI = 100000
_CHT = 1024
_HU = -(-(_NU // 2) // _CHT) * _CHT
_HI = -(-(_NI // 2) // _CHT) * _CHT
_CHB = 65536


def _half_offset(n):
  return pl.cdiv(n // 2, _CHT) * _CHT


def _pack_t(table_t):
  n = table_t.shape[1]
  h = _half_offset(n)
  nblk = h // _CHT
  maxblk = pl.cdiv(n, _CHT) - 1

  def body(a_ref, b_ref, eye_ref, o_ref):
    dn = (((0,), (0,)), ((), ()))
    eye = eye_ref[...]
    o_ref[:, 0:_D] = lax.dot_general(
        a_ref[...], eye, dn, preferred_element_type=jnp.float32)
    o_ref[:, _D:2 * _D] = lax.dot_general(
        b_ref[...], eye, dn, preferred_element_type=jnp.float32)

  return pl.pallas_call(
      body,
      grid=(nblk,),
      in_specs=[
          pl.BlockSpec((_D, _CHT), lambda i: (0, i)),
          pl.BlockSpec(
              (_D, _CHT),
              lambda i, _n=nblk, _m=maxblk: (0, jnp.minimum(i + _n, _m))),
          pl.BlockSpec((_D, _D), lambda i: (0, 0)),
      ],
      out_specs=pl.BlockSpec((_CHT, 2 * _D), lambda i: (i, 0)),
      out_shape=jax.ShapeDtypeStruct((h, 2 * _D), jnp.float32),
  )(table_t, table_t, jnp.eye(_D, dtype=jnp.float32))


def _bias_1d(bias_t):
  n = bias_t.shape[1]

  def body(x_ref, o_ref):
    o_ref[...] = x_ref[...].reshape(_CHB)

  return pl.pallas_call(
      body,
      grid=(pl.cdiv(n, _CHB),),
      in_specs=[pl.BlockSpec((1, _CHB), lambda i: (0, i))],
      out_specs=pl.BlockSpec((_CHB,), lambda i: (i,)),
      out_shape=jax.ShapeDtypeStruct((n,), jnp.float32),
  )(bias_t)


def _sc_body(uid_hbm, iid_hbm, uidm_hbm, iidm_hbm, uf_hbm, if_hbm, ub_hbm,
             ib_hbm, gb_hbm, out_hbm, uid_v, iid_v, uidm_v, iidm_v,
             urows_v, irows_v, ub_v, ib_v, gb_v, out_v, bsem, fsem):
  wid = lax.axis_index("s") * _NC + lax.axis_index("c")
  base = wid * _BPW
  pltpu.sync_copy(uid_hbm.at[pl.ds(base, _BPW)], uid_v)
  pltpu.sync_copy(iid_hbm.at[pl.ds(base, _BPW)], iid_v)
  pltpu.sync_copy(gb_hbm, gb_v)

  row0 = wid * _NWIN
  pltpu.sync_copy(uidm_hbm.at[pl.ds(row0, _NWIN)], uidm_v)
  pltpu.sync_copy(iidm_hbm.at[pl.ds(row0, _NWIN)], iidm_v)

  bias_copies = []
  for j in range(_NWIN):
    sl = pl.ds(j * _W, _W)
    bias_copies.append(
        pltpu.async_copy(ub_hbm.at[uid_v.at[sl]], ub_v.at[sl], bsem))
    bias_copies.append(
        pltpu.async_copy(ib_hbm.at[iid_v.at[sl]], ib_v.at[sl], bsem))

  def fire(w):
    buf = w % 2
    cu = pltpu.async_copy(uf_hbm.at[uidm_v.at[w]], urows_v.at[buf],
                          fsem.at[buf])
    ci = pltpu.async_copy(if_hbm.at[iidm_v.at[w]], irows_v.at[buf],
                          fsem.at[buf])
    return (cu, ci)

  lane = lax.iota(jnp.int32, _L)
  inflight = fire(0)
  for w in range(_NWIN):
    nxt = fire(w + 1) if w + 1 < _NWIN else None
    for c in inflight:
      c.wait()
    inflight = nxt
    buf = w % 2

    @pl.loop(0, _W // _L)
    def _(g):
      b0 = w * _W + g * _L
      r0 = g * _L
      pu16 = (uid_v[pl.ds(b0, _L)] >= _HU).astype(jnp.int32)
      pv16 = (iid_v[pl.ds(b0, _L)] >= _HI).astype(jnp.int32)
      acc = jnp.zeros((_L,), jnp.float32)
      for r in range(_L):
        pu = pu16[r] != 0
        pv = pv16[r] != 0
        p = jnp.zeros((_L,), jnp.float32)
        for k in range(_D // _L):
          uc = jnp.where(pu, urows_v[buf, r0 + r, pl.ds(_D + k * _L, _L)],
                         urows_v[buf, r0 + r, pl.ds(k * _L, _L)])
          vc = jnp.where(pv, irows_v[buf, r0 + r, pl.ds(_D + k * _L, _L)],
                         irows_v[buf, r0 + r, pl.ds(k * _L, _L)])
          p += uc * vc
        acc = jnp.where(lane == r, acc + jnp.sum(p), acc)
      out_v[pl.ds(b0, _L)] = acc

  for c in bias_copies:
    c.wait()

  gb16 = gb_v[...]

  @pl.loop(0, _BPW // _L)
  def _(g):
    sl = pl.ds(g * _L, _L)
    out_v[sl] = out_v[sl] + ub_v[sl] + ib_v[sl] + gb16

  pltpu.sync_copy(out_v, out_hbm.at[pl.ds(base, _BPW)])


def kernel(user_ids, item_ids, user_factors, item_factors, user_biases,
           item_biases, global_bias):
  mesh = plsc.VectorSubcoreMesh(core_axis_name="c", subcore_axis_name="s")
  cp = pltpu.CompilerParams(use_tc_tiling_on_sc=True)
  if "needs_layout_passes" in pltpu.CompilerParams.__dataclass_fields__:
    cp = dataclasses.replace(cp, needs_layout_passes=False)
  sc_call = pl.kernel(
      _sc_body,
      mesh=mesh,
      compiler_params=cp,
      out_type=jax.ShapeDtypeStruct((_B,), jnp.float32),
      scratch_types=[
          pltpu.VMEM((_BPW,), jnp.int32),
          pltpu.VMEM((_BPW,), jnp.int32),
          pltpu.VMEM((_NWIN, _W), jnp.int32),
          pltpu.VMEM((_NWIN, _W), jnp.int32),
          pltpu.VMEM((2, _W, 2 * _D), jnp.float32),
          pltpu.VMEM((2, _W, 2 * _D), jnp.float32),
          pltpu.VMEM((_BPW,), jnp.float32),
          pltpu.VMEM((_BPW,), jnp.float32),
          pltpu.VMEM((_L,), jnp.float32),
          pltpu.VMEM((_BPW,), jnp.float32),
          pltpu.SemaphoreType.DMA,
          pltpu.SemaphoreType.DMA((2,)),
      ],
  )
  uidm = jnp.where(user_ids >= _HU, user_ids - _HU, user_ids)
  iidm = jnp.where(item_ids >= _HI, item_ids - _HI, item_ids)
  return sc_call(
      user_ids,
      item_ids,
      uidm.reshape(_B // _W, _W),
      iidm.reshape(_B // _W, _W),
      _pack_t(user_factors.T),
      _pack_t(item_factors.T),
      _bias_1d(user_biases.T),
      _bias_1d(item_biases.T),
      jnp.broadcast_to(global_bias, (_L,)),
  )


# speedup vs baseline: 1.2325x; 1.2325x over previous
import dataclasses

import jax
import jax.numpy as jnp
from jax import lax
from jax.experimental import pallas as pl
from jax.experimental.pallas import tpu as pltpu
from jax.experimental.pallas import tpu_sc as plsc

_B = 16384
_D = 64
_L = 16
_NC = 2
_NS = 16
_NW = _NC * _NS
_BPW = _B // _NW
_W = 128
_NWIN = _BPW // _W
_RING = 6
_WAVE = 8
_NWAVES = _BPW // _WAVE


def _sc_body(uid_hbm, iid_hbm, uf_hbm, if_hbm, ub_hbm,
             ib_hbm, gb_hbm, out_hbm, uid_v, iid_v,
             ublk_v, iblk_v, ub_v, ib_v, gb_v, out_v, bsem, fsem):
  wid = lax.axis_index("s") * _NC + lax.axis_index("c")
  base = wid * _BPW
  pltpu.sync_copy(uid_hbm.at[pl.ds(base, _BPW)], uid_v.at[pl.ds(0, _BPW)])
  pltpu.sync_copy(iid_hbm.at[pl.ds(base, _BPW)], iid_v.at[pl.ds(0, _BPW)])
  pltpu.sync_copy(gb_hbm, gb_v)

  bias_copies = []
  for j in range(_NWIN):
    sl = pl.ds(j * _W, _W)
    bias_copies.append(
        pltpu.async_copy(ub_hbm.at[uid_v.at[sl]], ub_v.at[sl], bsem))
    bias_copies.append(
        pltpu.async_copy(ib_hbm.at[iid_v.at[sl]], ib_v.at[sl], bsem))

  def fire(g):
    ring = g % _RING
    uvec = uid_v[pl.ds(g * _WAVE, _L)]
    ivec = iid_v[pl.ds(g * _WAVE, _L)]
    ub8 = (uvec >> 3) << 3
    ib8 = (ivec >> 3) << 3
    for r in range(_WAVE):
      u0 = pl.multiple_of(ub8[r], 8)
      i0 = pl.multiple_of(ib8[r], 8)
      pltpu.async_copy(uf_hbm.at[pl.ds(u0, 8), :],
                       ublk_v.at[ring, r], fsem.at[ring])
      pltpu.async_copy(if_hbm.at[pl.ds(i0, 8), :],
                       iblk_v.at[ring, r], fsem.at[ring])

  def drain(g):
    ring = g % _RING
    for r in range(_WAVE):
      pltpu.make_async_copy(uf_hbm.at[pl.ds(0, 8), :],
                            ublk_v.at[ring, r], fsem.at[ring]).wait()
      pltpu.make_async_copy(if_hbm.at[pl.ds(0, 8), :],
                            iblk_v.at[ring, r], fsem.at[ring]).wait()

  lane = lax.iota(jnp.int32, _L)
  for g in range(_RING - 1):
    fire(g)

  @pl.loop(0, _NWAVES)
  def _(g):
    @pl.when(g + _RING - 1 < _NWAVES)
    def _():
      fire(g + _RING - 1)

    drain(g)
    ring = g % _RING
    b0 = g * _WAVE
    urem = uid_v[pl.ds(b0, _L)] & 7
    irem = iid_v[pl.ds(b0, _L)] & 7
    acc = jnp.zeros((_L,), jnp.float32)
    for r in range(_WAVE):
      ur = urem[r]
      ir = irem[r]
      p = (ublk_v[ring, r, ur, pl.ds(0, _L)] *
           iblk_v[ring, r, ir, pl.ds(0, _L)])
      for k in range(1, _D // _L):
        p += (ublk_v[ring, r, ur, pl.ds(k * _L, _L)] *
              iblk_v[ring, r, ir, pl.ds(k * _L, _L)])
      acc = jnp.where(lane == r, acc + jnp.sum(p), acc)
    out_v[pl.ds(b0, _L)] = acc

  for c in bias_copies:
    c.wait()

  gb16 = gb_v[...]

  @pl.loop(0, _BPW // _L)
  def _(g):
    sl = pl.ds(g * _L, _L)
    out_v[sl] = out_v[sl] + ub_v[sl] + ib_v[sl] + gb16

  pltpu.sync_copy(out_v, out_hbm.at[pl.ds(base, _BPW)])


_CHB = 262144


def _bias_1d(bias_t):
  n = bias_t.shape[1]

  def body(x_ref, o_ref):
    o_ref[...] = x_ref[...].reshape(_CHB)

  return pl.pallas_call(
      body,
      grid=(pl.cdiv(n, _CHB),),
      in_specs=[pl.BlockSpec((1, _CHB), lambda i: (0, i))],
      out_specs=pl.BlockSpec((_CHB,), lambda i: (i,)),
      out_shape=jax.ShapeDtypeStruct((n,), jnp.float32),
  )(bias_t)


def kernel(user_ids, item_ids, user_factors, item_factors, user_biases,
           item_biases, global_bias):
  mesh = plsc.VectorSubcoreMesh(core_axis_name="c", subcore_axis_name="s")
  cp = pltpu.CompilerParams(use_tc_tiling_on_sc=True)
  if "needs_layout_passes" in pltpu.CompilerParams.__dataclass_fields__:
    cp = dataclasses.replace(cp, needs_layout_passes=False)
  sc_call = pl.kernel(
      _sc_body,
      mesh=mesh,
      compiler_params=cp,
      out_type=jax.ShapeDtypeStruct((_B,), jnp.float32),
      scratch_types=[
          pltpu.VMEM((_BPW + _L,), jnp.int32),
          pltpu.VMEM((_BPW + _L,), jnp.int32),
          pltpu.VMEM((_RING, _WAVE, 8, _D), jnp.float32),
          pltpu.VMEM((_RING, _WAVE, 8, _D), jnp.float32),
          pltpu.VMEM((_BPW,), jnp.float32),
          pltpu.VMEM((_BPW,), jnp.float32),
          pltpu.VMEM((_L,), jnp.float32),
          pltpu.VMEM((_BPW,), jnp.float32),
          pltpu.SemaphoreType.DMA,
          pltpu.SemaphoreType.DMA((_RING,)),
      ],
  )
  return sc_call(
      user_ids,
      item_ids,
      user_factors,
      item_factors,
      _bias_1d(user_biases.T),
      _bias_1d(item_biases.T),
      jnp.broadcast_to(global_bias, (_L,)),
  )


# speedup vs baseline: 1.2348x; 1.0019x over previous
import dataclasses

import jax
import jax.numpy as jnp
from jax import lax
from jax.experimental import pallas as pl
from jax.experimental.pallas import tpu as pltpu
from jax.experimental.pallas import tpu_sc as plsc

_B = 16384
_D = 64
_L = 16
_NC = 2
_NS = 16
_NW = _NC * _NS
_BPW = _B // _NW
_W = 128
_NWIN = _BPW // _W
_RING = 6
_WAVE = 8
_NWAVES = _BPW // _WAVE


def _sc_body(uid_hbm, iid_hbm, uf_hbm, if_hbm, ub_hbm,
             ib_hbm, gb_hbm, dummy_hbm, out_hbm, uid_v, iid_v,
             ublk_v, iblk_v, ub_v, ib_v, gb_v, out_v, bsem, fsem):
  wid = lax.axis_index("s") * _NC + lax.axis_index("c")
  base = wid * _BPW
  pltpu.sync_copy(uid_hbm.at[pl.ds(base, _BPW)], uid_v.at[pl.ds(0, _BPW)])
  pltpu.sync_copy(iid_hbm.at[pl.ds(base, _BPW)], iid_v.at[pl.ds(0, _BPW)])
  pltpu.sync_copy(gb_hbm, gb_v)

  bias_copies = []
  for j in range(_NWIN):
    sl = pl.ds(j * _W, _W)
    bias_copies.append(
        pltpu.async_copy(ub_hbm.at[uid_v.at[sl]], ub_v.at[sl], bsem))
    bias_copies.append(
        pltpu.async_copy(ib_hbm.at[iid_v.at[sl]], ib_v.at[sl], bsem))

  def fire(g):
    ring = g % _RING
    uvec = uid_v[pl.ds(g * _WAVE, _L)]
    ivec = iid_v[pl.ds(g * _WAVE, _L)]
    ub8 = (uvec >> 3) << 3
    ib8 = (ivec >> 3) << 3
    for r in range(_WAVE):
      u0 = pl.multiple_of(ub8[r], 8)
      i0 = pl.multiple_of(ib8[r], 8)
      pltpu.async_copy(uf_hbm.at[pl.ds(u0, 8), :],
                       ublk_v.at[ring, r], fsem.at[ring])
      pltpu.async_copy(if_hbm.at[pl.ds(i0, 8), :],
                       iblk_v.at[ring, r], fsem.at[ring])

  def drain(g):
    ring = g % _RING
    pltpu.make_async_copy(dummy_hbm, ublk_v.at[ring], fsem.at[ring]).wait()
    pltpu.make_async_copy(dummy_hbm, iblk_v.at[ring], fsem.at[ring]).wait()

  lane = lax.iota(jnp.int32, _L)
  for g in range(_RING - 1):
    fire(g)

  @pl.loop(0, _NWAVES)
  def _(g):
    @pl.when(g + _RING - 1 < _NWAVES)
    def _():
      fire(g + _RING - 1)

    drain(g)
    ring = g % _RING
    b0 = g * _WAVE
    urem = uid_v[pl.ds(b0, _L)] & 7
    irem = iid_v[pl.ds(b0, _L)] & 7
    acc = jnp.zeros((_L,), jnp.float32)
    for r in range(_WAVE):
      ur = urem[r]
      ir = irem[r]
      p = (ublk_v[ring, r, ur, pl.ds(0, _L)] *
           iblk_v[ring, r, ir, pl.ds(0, _L)])
      for k in range(1, _D // _L):
        p += (ublk_v[ring, r, ur, pl.ds(k * _L, _L)] *
              iblk_v[ring, r, ir, pl.ds(k * _L, _L)])
      acc = jnp.where(lane == r, acc + jnp.sum(p), acc)
    out_v[pl.ds(b0, _L)] = acc

  for c in bias_copies:
    c.wait()

  gb16 = gb_v[...]

  @pl.loop(0, _BPW // _L)
  def _(g):
    sl = pl.ds(g * _L, _L)
    out_v[sl] = out_v[sl] + ub_v[sl] + ib_v[sl] + gb16

  pltpu.sync_copy(out_v, out_hbm.at[pl.ds(base, _BPW)])


_CHB = 262144


def _bias_1d(bias_t):
  n = bias_t.shape[1]

  def body(x_ref, o_ref):
    o_ref[...] = x_ref[...].reshape(_CHB)

  return pl.pallas_call(
      body,
      grid=(pl.cdiv(n, _CHB),),
      in_specs=[pl.BlockSpec((1, _CHB), lambda i: (0, i))],
      out_specs=pl.BlockSpec((_CHB,), lambda i: (i,)),
      out_shape=jax.ShapeDtypeStruct((n,), jnp.float32),
  )(bias_t)


def kernel(user_ids, item_ids, user_factors, item_factors, user_biases,
           item_biases, global_bias):
  mesh = plsc.VectorSubcoreMesh(core_axis_name="c", subcore_axis_name="s")
  cp = pltpu.CompilerParams(use_tc_tiling_on_sc=True)
  if "needs_layout_passes" in pltpu.CompilerParams.__dataclass_fields__:
    cp = dataclasses.replace(cp, needs_layout_passes=False)
  sc_call = pl.kernel(
      _sc_body,
      mesh=mesh,
      compiler_params=cp,
      out_type=jax.ShapeDtypeStruct((_B,), jnp.float32),
      scratch_types=[
          pltpu.VMEM((_BPW + _L,), jnp.int32),
          pltpu.VMEM((_BPW + _L,), jnp.int32),
          pltpu.VMEM((_RING, _WAVE, 8, _D), jnp.float32),
          pltpu.VMEM((_RING, _WAVE, 8, _D), jnp.float32),
          pltpu.VMEM((_BPW,), jnp.float32),
          pltpu.VMEM((_BPW,), jnp.float32),
          pltpu.VMEM((_L,), jnp.float32),
          pltpu.VMEM((_BPW,), jnp.float32),
          pltpu.SemaphoreType.DMA,
          pltpu.SemaphoreType.DMA((_RING,)),
      ],
  )
  return sc_call(
      user_ids,
      item_ids,
      user_factors,
      item_factors,
      _bias_1d(user_biases.T),
      _bias_1d(item_biases.T),
      jnp.broadcast_to(global_bias, (_L,)),
      jnp.zeros((_WAVE, 8, _D), jnp.float32),
  )
